# baseline (speedup 1.0000x reference)
import jax
import jax.numpy as jnp
from jax.experimental import pallas as pl

_INTERPRET = False

_RANK_CHUNK = 256


def _scores_body(nf_ref, noise_ref, w_ref, b_ref, s_ref, min_ref):
    x = nf_ref[0] + 0.05 * noise_ref[0]
    s = jnp.sum(x * w_ref[...], axis=1, keepdims=True)
    s = s + b_ref[0, 0]
    s_ref[0] = s
    min_ref[...] = jnp.min(s).reshape(1, 1, 1)


def _prefix_sum_row(x, lane):
    n = x.shape[1]
    d = 1
    while d < n:
        shifted = jnp.roll(x, d, axis=1)
        x = x + jnp.where(lane >= d, shifted, 0.0)
        d *= 2
    return x


def _sort_body(scol_ref, srow_ref, minv_ref, mcol_ref, mrow_ref,
               u_ref, ru_ref, d_ref, v_ref, rv_ref):
    n = scol_ref.shape[1]
    fill = jnp.min(minv_ref[...]) - 1.0
    mcol = mcol_ref[0] != 0
    scol = jnp.where(mcol, scol_ref[0], fill)
    srow = jnp.where(mrow_ref[0] != 0, srow_ref[0], fill)

    ch = _RANK_CHUNK
    acc = jnp.zeros((1, n), jnp.float32)
    col_iota = jax.lax.broadcasted_iota(jnp.int32, (ch, n), 1)
    for c in range(n // ch):
        sj = jax.lax.slice(scol, (c * ch, 0), ((c + 1) * ch, 1))
        row_iota = jax.lax.broadcasted_iota(jnp.int32, (ch, n), 0) + c * ch
        gt = srow > sj
        tie = (srow == sj) & (col_iota < row_iota)
        rank = jnp.sum((gt | tie).astype(jnp.int32), axis=1, keepdims=True)
        onehot = col_iota == rank
        acc = acc + jnp.sum(jnp.where(onehot, sj, 0.0), axis=0, keepdims=True)
    ss = acc

    c0 = (jnp.max(scol) + jnp.min(scol)) * 0.5

    mf = mcol.astype(jnp.float32)
    eb_col = jnp.exp(scol - c0)
    u_ref[0] = mf * eb_col
    ru_ref[0] = mf / eb_col
    d_ref[0] = 1.0 - mf

    b_row = jnp.exp(ss - c0)
    a_row = 1.0 / b_row
    lane = jax.lax.broadcasted_iota(jnp.int32, (1, n), 1)
    pa = _prefix_sum_row(a_row, lane)
    pb = _prefix_sum_row(b_row, lane)
    bt = jnp.sum(b_row)
    denom = b_row * pa + a_row * (bt - pb)
    rd = 1.0 / denom
    v_ref[0] = b_row * rd
    rv_ref[0] = a_row * rd


def _perm_body(u_ref, ru_ref, d_ref, v_ref, rv_ref, out_ref):
    n, ibk = out_ref.shape[1], out_ref.shape[2]
    ib = pl.program_id(1)
    vrow = v_ref[0]
    rvrow = rv_ref[0]
    nq = n // ibk
    eye = (jax.lax.broadcasted_iota(jnp.int32, (ibk, ibk), 0) ==
           jax.lax.broadcasted_iota(jnp.int32, (ibk, ibk), 1))
    for q in range(nq):
        sl = pl.ds(q * ibk, ibk)
        ucol = u_ref[0, sl, :]
        rucol = ru_ref[0, sl, :]
        p = jnp.minimum(ucol * rvrow, rucol * vrow)

        @pl.when(ib == q)
        def _():
            out_ref[0, sl, :] = p + jnp.where(eye, d_ref[0, sl, :], 0.0)

        @pl.when(ib != q)
        def _():
            out_ref[0, sl, :] = p


def kernel(node_features, mask, W, b, noise):
    B, N, D = node_features.shape
    mask_i = mask.astype(jnp.int32)
    w_row = W.reshape(1, D)
    b2 = b.reshape(1, 1)

    scores_col, minv = pl.pallas_call(
        _scores_body,
        grid=(B,),
        in_specs=[
            pl.BlockSpec((1, N, D), lambda i: (i, 0, 0)),
            pl.BlockSpec((1, N, D), lambda i: (i, 0, 0)),
            pl.BlockSpec((1, D), lambda i: (0, 0)),
            pl.BlockSpec((1, 1), lambda i: (0, 0)),
        ],
        out_specs=[
            pl.BlockSpec((1, N, 1), lambda i: (i, 0, 0)),
            pl.BlockSpec((1, 1, 1), lambda i: (i, 0, 0)),
        ],
        out_shape=[
            jax.ShapeDtypeStruct((B, N, 1), jnp.float32),
            jax.ShapeDtypeStruct((B, 1, 1), jnp.float32),
        ],
        interpret=_INTERPRET,
    )(node_features, noise, w_row, b2)

    scores_row = scores_col.reshape(B, 1, N)
    mask_col = mask_i.reshape(B, N, 1)
    mask_row = mask_i.reshape(B, 1, N)

    ucol, rucol, dcol, vrow, rvrow = pl.pallas_call(
        _sort_body,
        grid=(B,),
        in_specs=[
            pl.BlockSpec((1, N, 1), lambda i: (i, 0, 0)),
            pl.BlockSpec((1, 1, N), lambda i: (i, 0, 0)),
            pl.BlockSpec((B, 1, 1), lambda i: (0, 0, 0)),
            pl.BlockSpec((1, N, 1), lambda i: (i, 0, 0)),
            pl.BlockSpec((1, 1, N), lambda i: (i, 0, 0)),
        ],
        out_specs=[
            pl.BlockSpec((1, N, 1), lambda i: (i, 0, 0)),
            pl.BlockSpec((1, N, 1), lambda i: (i, 0, 0)),
            pl.BlockSpec((1, N, 1), lambda i: (i, 0, 0)),
            pl.BlockSpec((1, 1, N), lambda i: (i, 0, 0)),
            pl.BlockSpec((1, 1, N), lambda i: (i, 0, 0)),
        ],
        out_shape=[
            jax.ShapeDtypeStruct((B, N, 1), jnp.float32),
            jax.ShapeDtypeStruct((B, N, 1), jnp.float32),
            jax.ShapeDtypeStruct((B, N, 1), jnp.float32),
            jax.ShapeDtypeStruct((B, 1, N), jnp.float32),
            jax.ShapeDtypeStruct((B, 1, N), jnp.float32),
        ],
        interpret=_INTERPRET,
    )(scores_col, scores_row, minv, mask_col, mask_row)

    IBK = 512
    out = pl.pallas_call(
        _perm_body,
        grid=(B, N // IBK),
        in_specs=[
            pl.BlockSpec((1, N, 1), lambda bb, ib: (bb, 0, 0)),
            pl.BlockSpec((1, N, 1), lambda bb, ib: (bb, 0, 0)),
            pl.BlockSpec((1, N, 1), lambda bb, ib: (bb, 0, 0)),
            pl.BlockSpec((1, 1, IBK), lambda bb, ib: (bb, 0, ib)),
            pl.BlockSpec((1, 1, IBK), lambda bb, ib: (bb, 0, ib)),
        ],
        out_specs=pl.BlockSpec((1, N, IBK), lambda bb, ib: (bb, 0, ib)),
        out_shape=jax.ShapeDtypeStruct((B, N, N), jnp.float32),
        interpret=_INTERPRET,
    )(ucol, rucol, dcol, vrow, rvrow)
    return out


# speedup vs baseline: 1.0251x; 1.0251x over previous
import jax
import jax.numpy as jnp
from jax.experimental import pallas as pl

_INTERPRET = False

_RANK_CHUNK = 256


def _scores_body(nf_ref, noise_ref, w_ref, b_ref, s_ref, min_ref):
    x = nf_ref[0] + 0.05 * noise_ref[0]
    s = jnp.sum(x * w_ref[...], axis=1, keepdims=True)
    s = s + b_ref[0, 0]
    s_ref[0] = s
    min_ref[...] = jnp.min(s).reshape(1, 1, 1)


def _prefix_sum_row(x, lane):
    n = x.shape[1]
    d = 1
    while d < n:
        shifted = jnp.roll(x, d, axis=1)
        x = x + jnp.where(lane >= d, shifted, 0.0)
        d *= 2
    return x


def _sort_body(scol_ref, srow_ref, minv_ref, mcol_ref, mrow_ref,
               u_ref, ru_ref, d_ref, v_ref, rv_ref):
    n = scol_ref.shape[1]
    fill = jnp.min(minv_ref[...]) - 1.0
    mcol = mcol_ref[0] != 0
    scol = jnp.where(mcol, scol_ref[0], fill)
    srow = jnp.where(mrow_ref[0] != 0, srow_ref[0], fill)

    ch = _RANK_CHUNK
    acc = jnp.zeros((1, n), jnp.float32)
    col_iota = jax.lax.broadcasted_iota(jnp.int32, (ch, n), 1)
    for c in range(n // ch):
        sj = jax.lax.slice(scol, (c * ch, 0), ((c + 1) * ch, 1))
        row_iota = jax.lax.broadcasted_iota(jnp.int32, (ch, n), 0) + c * ch
        gt = srow > sj
        tie = (srow == sj) & (col_iota < row_iota)
        rank = jnp.sum((gt | tie).astype(jnp.int32), axis=1, keepdims=True)
        onehot = col_iota == rank
        acc = acc + jnp.sum(jnp.where(onehot, sj, 0.0), axis=0, keepdims=True)
    ss = acc

    c0 = (jnp.max(scol) + jnp.min(scol)) * 0.5

    mf = mcol.astype(jnp.float32)
    eb_col = jnp.exp(scol - c0)
    u_ref[0] = mf * eb_col
    ru_ref[0] = mf / eb_col
    d_ref[0] = 1.0 - mf

    b_row = jnp.exp(ss - c0)
    a_row = 1.0 / b_row
    lane = jax.lax.broadcasted_iota(jnp.int32, (1, n), 1)
    pa = _prefix_sum_row(a_row, lane)
    pb = _prefix_sum_row(b_row, lane)
    bt = jnp.sum(b_row)
    denom = b_row * pa + a_row * (bt - pb)
    rd = 1.0 / denom
    v_ref[0] = b_row * rd
    rv_ref[0] = a_row * rd


def _perm_body(u_ref, ru_ref, d_ref, v_ref, rv_ref, out_ref):
    n, ibk = out_ref.shape[1], out_ref.shape[2]
    ib = pl.program_id(1)
    vrow = v_ref[0]
    rvrow = rv_ref[0]
    nq = n // ibk
    for q in range(nq):
        sl = pl.ds(q * ibk, ibk)

        @pl.when(ib == q)
        def _():
            eye = (jax.lax.broadcasted_iota(jnp.int32, (ibk, ibk), 0) ==
                   jax.lax.broadcasted_iota(jnp.int32, (ibk, ibk), 1))
            p = jnp.minimum(u_ref[0, sl, :] * rvrow, ru_ref[0, sl, :] * vrow)
            out_ref[0, sl, :] = jnp.where(eye, p + d_ref[0, sl, :], p)

        @pl.when(ib != q)
        def _():
            out_ref[0, sl, :] = jnp.minimum(u_ref[0, sl, :] * rvrow,
                                            ru_ref[0, sl, :] * vrow)


def kernel(node_features, mask, W, b, noise):
    B, N, D = node_features.shape
    mask_i = mask.astype(jnp.int32)
    w_row = W.reshape(1, D)
    b2 = b.reshape(1, 1)

    scores_col, minv = pl.pallas_call(
        _scores_body,
        grid=(B,),
        in_specs=[
            pl.BlockSpec((1, N, D), lambda i: (i, 0, 0)),
            pl.BlockSpec((1, N, D), lambda i: (i, 0, 0)),
            pl.BlockSpec((1, D), lambda i: (0, 0)),
            pl.BlockSpec((1, 1), lambda i: (0, 0)),
        ],
        out_specs=[
            pl.BlockSpec((1, N, 1), lambda i: (i, 0, 0)),
            pl.BlockSpec((1, 1, 1), lambda i: (i, 0, 0)),
        ],
        out_shape=[
            jax.ShapeDtypeStruct((B, N, 1), jnp.float32),
            jax.ShapeDtypeStruct((B, 1, 1), jnp.float32),
        ],
        interpret=_INTERPRET,
    )(node_features, noise, w_row, b2)

    scores_row = scores_col.reshape(B, 1, N)
    mask_col = mask_i.reshape(B, N, 1)
    mask_row = mask_i.reshape(B, 1, N)

    ucol, rucol, dcol, vrow, rvrow = pl.pallas_call(
        _sort_body,
        grid=(B,),
        in_specs=[
            pl.BlockSpec((1, N, 1), lambda i: (i, 0, 0)),
            pl.BlockSpec((1, 1, N), lambda i: (i, 0, 0)),
            pl.BlockSpec((B, 1, 1), lambda i: (0, 0, 0)),
            pl.BlockSpec((1, N, 1), lambda i: (i, 0, 0)),
            pl.BlockSpec((1, 1, N), lambda i: (i, 0, 0)),
        ],
        out_specs=[
            pl.BlockSpec((1, N, 1), lambda i: (i, 0, 0)),
            pl.BlockSpec((1, N, 1), lambda i: (i, 0, 0)),
            pl.BlockSpec((1, N, 1), lambda i: (i, 0, 0)),
            pl.BlockSpec((1, 1, N), lambda i: (i, 0, 0)),
            pl.BlockSpec((1, 1, N), lambda i: (i, 0, 0)),
        ],
        out_shape=[
            jax.ShapeDtypeStruct((B, N, 1), jnp.float32),
            jax.ShapeDtypeStruct((B, N, 1), jnp.float32),
            jax.ShapeDtypeStruct((B, N, 1), jnp.float32),
            jax.ShapeDtypeStruct((B, 1, N), jnp.float32),
            jax.ShapeDtypeStruct((B, 1, N), jnp.float32),
        ],
        interpret=_INTERPRET,
    )(scores_col, scores_row, minv, mask_col, mask_row)

    IBK = 512
    out = pl.pallas_call(
        _perm_body,
        grid=(B, N // IBK),
        in_specs=[
            pl.BlockSpec((1, N, 1), lambda bb, ib: (bb, 0, 0)),
            pl.BlockSpec((1, N, 1), lambda bb, ib: (bb, 0, 0)),
            pl.BlockSpec((1, N, 1), lambda bb, ib: (bb, 0, 0)),
            pl.BlockSpec((1, 1, IBK), lambda bb, ib: (bb, 0, ib)),
            pl.BlockSpec((1, 1, IBK), lambda bb, ib: (bb, 0, ib)),
        ],
        out_specs=pl.BlockSpec((1, N, IBK), lambda bb, ib: (bb, 0, ib)),
        out_shape=jax.ShapeDtypeStruct((B, N, N), jnp.float32),
        interpret=_INTERPRET,
    )(ucol, rucol, dcol, vrow, rvrow)
    return out
